# baseline (speedup 1.0000x reference)
import functools

import jax
import jax.numpy as jnp
from jax import lax
from jax.experimental import pallas as pl
from jax.experimental.pallas import tpu as pltpu
from jax.experimental.pallas import tpu_sc as plsc

_NUM_STYLES = 64
_DIM = 192
_BATCH = 16384
_LANES = 16
_TW = _NUM_STYLES * _DIM


def _table_body(emb_ref, w_ref, b_ref, p_ref):
    p = (
        lax.dot_general(
            emb_ref[...],
            w_ref[...],
            (((1,), (1,)), ((), ())),
            preferred_element_type=jnp.float32,
        )
        + b_ref[...]
    )
    pv = p.reshape(32, 2, _DIM)
    pe = pv[:, 0, :]
    po = pv[:, 1, :]
    row_a = pe[:, :128]
    row_b = jnp.concatenate([pe[:, 128:], po[:, :64]], axis=1)
    row_c = po[:, 64:]
    p_ref[...] = jnp.stack([row_a, row_b, row_c], axis=1).reshape(96, 128)


def _assemble_body(lin_ref, exa_ref, out_ref):
    v = lin_ref[...].reshape(256, 3, 128)
    a = v[:, 0, :]
    b = v[:, 1, :]
    c = v[:, 2, :]
    even = jnp.concatenate([a, b[:, :64]], axis=1)
    odd = jnp.concatenate([b[:, 64:], c], axis=1)
    rows = jnp.stack([even, odd], axis=1).reshape(512, _DIM)
    out_ref[...] = rows * exa_ref[...]


def _make_sc_kernel():
    info = plsc.get_sparse_core_info()
    nc, ns = info.num_cores, info.num_subcores
    nw = nc * ns
    bpw = _BATCH // nw
    nch = 4
    ch = bpw // nch
    ng = ch // _LANES
    nvec = _DIM // _LANES

    mesh = plsc.VectorSubcoreMesh(core_axis_name="c", subcore_axis_name="s")

    @functools.partial(
        pl.kernel,
        mesh=mesh,
        compiler_params=pltpu.CompilerParams(
            needs_layout_passes=False, use_tc_tiling_on_sc=False
        ),
        out_type=jax.ShapeDtypeStruct((_BATCH * _DIM,), jnp.float32),
        scratch_types=[
            pltpu.VMEM((_TW,), jnp.float32),
            pltpu.VMEM((bpw,), jnp.int32),
            pltpu.VMEM((2, ch * _DIM), jnp.float32),
            pltpu.SemaphoreType.DMA,
            pltpu.SemaphoreType.DMA,
        ],
    )
    def sc_kernel(sid_hbm, p_hbm, out_hbm, p_v, idx_v, sbuf, o0, o1):
        wid = lax.axis_index("s") * nc + lax.axis_index("c")
        base = wid * bpw
        osems = (o0, o1)
        pltpu.sync_copy(sid_hbm.at[wid], idx_v)
        pltpu.sync_copy(p_hbm, p_v)
        lane = lax.iota(jnp.int32, _LANES)
        cvecs = [lane + j * _LANES for j in range(nvec)]

        stores = [None, None]
        for c in range(nch):
            s = c % 2
            if stores[s] is not None:
                stores[s].wait()

            def group(g, _):
                off = c * ch + g * _LANES
                for rr in range(_LANES):
                    row = plsc.load_gather(
                        idx_v, [jnp.full((_LANES,), off + rr, jnp.int32)]
                    )
                    fb = row * _DIM
                    rloc = g * _LANES + rr
                    for j in range(nvec):
                        val = plsc.load_gather(p_v, [fb + cvecs[j]])
                        sbuf[s, pl.ds(rloc * _DIM + j * _LANES, _LANES)] = val
                return _

            lax.fori_loop(0, ng, group, 0)
            stores[s] = pltpu.async_copy(
                sbuf.at[s],
                out_hbm.at[pl.ds((base + c * ch) * _DIM, ch * _DIM)],
                osems[s],
            )
        for st in stores:
            st.wait()

    return sc_kernel, nw


_SC_KERNEL, _NW = _make_sc_kernel()


def kernel(style_id, exaggeration, emb, W, b):
    p96 = pl.pallas_call(
        _table_body,
        out_shape=jax.ShapeDtypeStruct((96, 128), jnp.float32),
    )(emb, W, b.reshape(1, _DIM))
    sid = style_id.reshape(_NW, _BATCH // _NW)
    flat = _SC_KERNEL(sid, p96.reshape(_TW))
    lin = flat.reshape(_BATCH * _DIM // 128, 128)
    nblk = 32
    rows_per_blk = _BATCH // nblk
    lin_rows = rows_per_blk * _DIM // 128
    return pl.pallas_call(
        _assemble_body,
        grid=(nblk,),
        in_specs=[
            pl.BlockSpec((lin_rows, 128), lambda i: (i, 0)),
            pl.BlockSpec((rows_per_blk, 1), lambda i: (i, 0)),
        ],
        out_specs=pl.BlockSpec((rows_per_blk, _DIM), lambda i: (i, 0)),
        out_shape=jax.ShapeDtypeStruct((_BATCH, _DIM), jnp.float32),
    )(lin, exaggeration.reshape(_BATCH, 1))
